# Initial kernel scaffold; baseline (speedup 1.0000x reference)
import jax
import jax.numpy as jnp
from jax.experimental import pallas as pl


def kernel(nodes, edge_index, W_ne, b_ne, W_ee, b_ee, Wn1, bn1, Wn2, bn2, We1, be1, We2, be2, Wp, bp):
    raise NotImplementedError("write your pallas kernel here")


# speedup vs baseline: 2.6324x; 2.6324x over previous
import functools

import jax
import jax.numpy as jnp
from jax import lax
from jax.experimental import pallas as pl
from jax.experimental.pallas import tpu as pltpu
from jax.experimental.pallas import tpu_sc as plsc

N = 10000
E = 320000
D = 128
NC = 2
NS = 16
NW = NC * NS
EPT = E // NW
K = 80
NCHUNK = EPT // K
RPT = 632
RPT_LAST = N - (NS - 1) * RPT

f32 = jnp.float32

_RN = 2000
_RE = 4000


def _full(shape):
    return pl.BlockSpec(shape, lambda i: (0,) * len(shape))


def _rows(shape):
    return pl.BlockSpec(shape, lambda i: (i,) + (0,) * (len(shape) - 1))


def _encode_body(x_ref, wne, bne, wees, weed, bee, we1a, we1b, be1,
                 n_ref, a_ref, b_ref, as_ref, ad_ref):
    x = x_ref[...]
    n = jnp.dot(x, wne[...], preferred_element_type=f32) + bne[...]
    n_ref[...] = n
    a_ref[...] = jnp.dot(n, wees[...], preferred_element_type=f32) + bee[...]
    b_ref[...] = jnp.dot(n, weed[...], preferred_element_type=f32)
    as_ref[...] = jnp.dot(n, we1a[...], preferred_element_type=f32) + be1[...]
    ad_ref[...] = jnp.dot(n, we1b[...], preferred_element_type=f32)


def _encode_call(nodes, wne, bne, wees, weed, bee, we1a, we1b, be1):
    w = _full((D, D))
    b = _full((1, D))
    r = _rows((_RN, D))
    return pl.pallas_call(
        _encode_body,
        grid=(N // _RN,),
        in_specs=[r, w, b, w, w, b, w, w, b],
        out_specs=[r, r, r, r, r],
        out_shape=[jax.ShapeDtypeStruct((N, D), f32)] * 5,
    )(nodes, wne, bne, wees, weed, bee, we1a, we1b, be1)


def _node_body(n_ref, m0_ref, m1_ref, wn1a, wn1b, bn1, wn2, bn2,
               we1a, we1b, be1, wps, wpd, bp,
               nn_ref, as_ref, ad_ref, ps_ref, pd_ref):
    ncur = n_ref[...]
    msg = m0_ref[...] + m1_ref[...]
    h = jnp.maximum(
        jnp.dot(ncur, wn1a[...], preferred_element_type=f32)
        + jnp.dot(msg, wn1b[...], preferred_element_type=f32) + bn1[...], 0.0)
    nn = jnp.dot(h, wn2[...], preferred_element_type=f32) + bn2[...] + ncur
    nn_ref[...] = nn
    as_ref[...] = jnp.dot(nn, we1a[...], preferred_element_type=f32) + be1[...]
    ad_ref[...] = jnp.dot(nn, we1b[...], preferred_element_type=f32)
    ps_ref[...] = jnp.sum(nn * wps[...], axis=1, keepdims=True) + bp[...]
    pd_ref[...] = jnp.sum(nn * wpd[...], axis=1, keepdims=True)


def _node_call(n, m0, m1, wn1a, wn1b, bn1, wn2, bn2, we1a, we1b, be1,
               wps, wpd, bp):
    w = _full((D, D))
    b = _full((1, D))
    r = _rows((_RN, D))
    r1 = _rows((_RN, 1))
    return pl.pallas_call(
        _node_body,
        grid=(N // _RN,),
        in_specs=[r, r, r, w, w, b, w, b, w, w, b, b, b, _full((1, 1))],
        out_specs=[r, r, r, r1, r1],
        out_shape=[jax.ShapeDtypeStruct((N, D), f32)] * 3
        + [jax.ShapeDtypeStruct((N, 1), f32)] * 2,
    )(n, m0, m1, wn1a, wn1b, bn1, wn2, bn2, we1a, we1b, be1, wps, wpd, bp)


def _edge_body(e_ref, g_ref, we1c, we2, be2, wpe, en_ref, ped_ref):
    e = e_ref[...]
    h = jnp.maximum(g_ref[...] + jnp.dot(e, we1c[...],
                                         preferred_element_type=f32), 0.0)
    en = jnp.dot(h, we2[...], preferred_element_type=f32) + be2[...] + e
    en_ref[...] = en
    ped_ref[...] = jnp.sum(en * wpe[...], axis=1, keepdims=True)


def _edge_call(e, g, we1c, we2, be2, wpe):
    w = _full((D, D))
    b = _full((1, D))
    r = _rows((_RE, D))
    return pl.pallas_call(
        _edge_body,
        grid=(E // _RE,),
        in_specs=[r, r, w, w, b, b],
        out_specs=[r, _rows((_RE, 1))],
        out_shape=[jax.ShapeDtypeStruct((E, D), f32),
                   jax.ShapeDtypeStruct((E, 1), f32)],
    )(e, g, we1c, we2, be2, wpe)


_MESH = plsc.VectorSubcoreMesh(core_axis_name="c", subcore_axis_name="s")


def _tile_slice_copy(src_ref, dst_ref, s):
    start = pl.multiple_of(s * RPT, 8)

    @pl.when(s < NS - 1)
    def _():
        pltpu.sync_copy(src_ref.at[pl.ds(start, RPT)],
                        dst_ref.at[pl.ds(start, RPT)])

    @pl.when(s == NS - 1)
    def _():
        pltpu.sync_copy(src_ref.at[pl.ds((NS - 1) * RPT, RPT_LAST)],
                        dst_ref.at[pl.ds((NS - 1) * RPT, RPT_LAST)])


def _fused_common(init, refs):
    if init:
        (a_hbm, b_hbm, as_hbm, ad_hbm, src_hbm, dst_hbm, zeros_hbm,
         e_out, g_out, m0_out, m1_out,
         idxs, idxd, bufa, bufb, bufea, bufeb, msg_sp,
         sema, semb, semc, semd) = refs
    else:
        (e_hbm, as_hbm, ad_hbm, src_hbm, dst_hbm, zeros_hbm,
         g_out, m0_out, m1_out,
         idxs, idxd, bufa, bufb, bufe, msg_sp, sema, semb) = refs
    c = lax.axis_index("c")
    s = lax.axis_index("s")
    wid = c * NS + s
    _tile_slice_copy(zeros_hbm, msg_sp, s)
    plsc.subcore_barrier()

    def chunk(j, carry):
        base = wid * EPT + j * K
        pltpu.sync_copy(src_hbm.at[pl.ds(base, K)], idxs)
        pltpu.sync_copy(dst_hbm.at[pl.ds(base, K)], idxd)
        cpa = pltpu.async_copy(as_hbm.at[idxs], bufa, sema)
        cpb = pltpu.async_copy(ad_hbm.at[idxd], bufb, semb)
        if init:
            cpea = pltpu.async_copy(a_hbm.at[idxs], bufea, semc)
            cpeb = pltpu.async_copy(b_hbm.at[idxd], bufeb, semd)
        else:
            pltpu.sync_copy(e_hbm.at[pl.ds(base, K)], bufe)
        cpa.wait()
        cpb.wait()
        if init:
            cpea.wait()
            cpeb.wait()

        def row(r, carry2):
            for u in range(D // 16):
                sl = pl.ds(u * 16, 16)
                bufa[r, sl] = bufa[r, sl] + bufb[r, sl]
                if init:
                    bufea[r, sl] = bufea[r, sl] + bufeb[r, sl]
            return carry2

        lax.fori_loop(0, K, row, 0)
        ebuf = bufea if init else bufe
        pltpu.sync_copy(ebuf, msg_sp.at[idxd], add=True)
        pltpu.sync_copy(bufa, g_out.at[pl.ds(base, K)])
        if init:
            pltpu.sync_copy(ebuf, e_out.at[pl.ds(base, K)])
        return carry

    lax.fori_loop(0, NCHUNK, chunk, 0)
    plsc.subcore_barrier()

    @pl.when(c == 0)
    def _():
        _tile_slice_copy(msg_sp, m0_out, s)

    @pl.when(c == 1)
    def _():
        _tile_slice_copy(msg_sp, m1_out, s)


def _fused_init_body(*refs):
    _fused_common(True, refs)


def _fused_iter_body(*refs):
    _fused_common(False, refs)


_ND = jax.ShapeDtypeStruct((N, D), f32)
_ED = jax.ShapeDtypeStruct((E, D), f32)

_fused_init = pl.kernel(
    _fused_init_body,
    out_type=[_ED, _ED, _ND, _ND],
    mesh=_MESH,
    scratch_types=[
        pltpu.VMEM((K,), jnp.int32),
        pltpu.VMEM((K,), jnp.int32),
        pltpu.VMEM((K, D), f32),
        pltpu.VMEM((K, D), f32),
        pltpu.VMEM((K, D), f32),
        pltpu.VMEM((K, D), f32),
        pltpu.VMEM_SHARED((N, D), f32),
        pltpu.SemaphoreType.DMA,
        pltpu.SemaphoreType.DMA,
        pltpu.SemaphoreType.DMA,
        pltpu.SemaphoreType.DMA,
    ],
)

_fused_iter = pl.kernel(
    _fused_iter_body,
    out_type=[_ED, _ND, _ND],
    mesh=_MESH,
    scratch_types=[
        pltpu.VMEM((K,), jnp.int32),
        pltpu.VMEM((K,), jnp.int32),
        pltpu.VMEM((K, D), f32),
        pltpu.VMEM((K, D), f32),
        pltpu.VMEM((K, D), f32),
        pltpu.VMEM_SHARED((N, D), f32),
        pltpu.SemaphoreType.DMA,
        pltpu.SemaphoreType.DMA,
    ],
)


def _pred_body(ps_hbm, pd_hbm, ped_hbm, src_hbm, dst_hbm, out_hbm,
               psv, pdv, idxs, idxd, pedv, outv):
    c = lax.axis_index("c")
    s = lax.axis_index("s")
    wid = c * NS + s
    base = wid * EPT
    pltpu.sync_copy(ps_hbm, psv)
    pltpu.sync_copy(pd_hbm, pdv)
    pltpu.sync_copy(src_hbm.at[pl.ds(base, EPT)], idxs)
    pltpu.sync_copy(dst_hbm.at[pl.ds(base, EPT)], idxd)
    pltpu.sync_copy(ped_hbm.at[pl.ds(base, EPT)], pedv)

    def grp(g, carry):
        sl = pl.ds(g * 16, 16)
        a = plsc.load_gather(psv, [idxs[sl]])
        b = plsc.load_gather(pdv, [idxd[sl]])
        outv[sl] = a + b + pedv[sl]
        return carry

    lax.fori_loop(0, EPT // 16, grp, 0)
    pltpu.sync_copy(outv, out_hbm.at[pl.ds(base, EPT)])


_pred = pl.kernel(
    _pred_body,
    out_type=jax.ShapeDtypeStruct((E,), f32),
    mesh=_MESH,
    compiler_params=pltpu.CompilerParams(needs_layout_passes=False),
    scratch_types=[
        pltpu.VMEM((N,), f32),
        pltpu.VMEM((N,), f32),
        pltpu.VMEM((EPT,), jnp.int32),
        pltpu.VMEM((EPT,), jnp.int32),
        pltpu.VMEM((EPT,), f32),
        pltpu.VMEM((EPT,), f32),
    ],
)


def kernel(nodes, edge_index, W_ne, b_ne, W_ee, b_ee, Wn1, bn1, Wn2, bn2,
           We1, be1, We2, be2, Wp, bp):
    src = edge_index[0]
    dst = edge_index[1]
    bne = b_ne.reshape(1, D)
    bee = b_ee.reshape(1, D)
    bn1r = bn1.reshape(1, D)
    bn2r = bn2.reshape(1, D)
    be1r = be1.reshape(1, D)
    be2r = be2.reshape(1, D)
    wees, weed = W_ee[:D], W_ee[D:]
    wn1a, wn1b = Wn1[:D], Wn1[D:]
    we1a, we1b, we1c = We1[:D], We1[D:2 * D], We1[2 * D:]
    wps = Wp[:D, 0].reshape(1, D)
    wpd = Wp[D:2 * D, 0].reshape(1, D)
    wpe = Wp[2 * D:, 0].reshape(1, D)
    bpr = bp.reshape(1, 1)
    zeros = jnp.zeros((N, D), f32)

    n, A, B, As, Ad = _encode_call(nodes, W_ne, bne, wees, weed, bee,
                                   we1a, we1b, be1r)
    e, G, m0, m1 = _fused_init(A, B, As, Ad, src, dst, zeros)
    ps = pd = ped = None
    for t in range(3):
        if t > 0:
            G, m0, m1 = _fused_iter(e, As, Ad, src, dst, zeros)
        n, As, Ad, ps, pd = _node_call(n, m0, m1, wn1a, wn1b, bn1r, Wn2,
                                       bn2r, we1a, we1b, be1r, wps, wpd, bpr)
        e, ped = _edge_call(e, G, we1c, We2, be2r, wpe)
    out = _pred(ps.reshape(N), pd.reshape(N), ped.reshape(E), src, dst)
    return out
